# baseline (speedup 1.0000x reference)
import functools

import jax
import jax.numpy as jnp
from jax import lax
from jax.experimental import pallas as pl
from jax.experimental.pallas import tpu as pltpu
from jax.experimental.pallas import tpu_sc as plsc


def _sc_geometry():
    try:
        info = plsc.get_sparse_core_info()
        return info.num_cores, info.num_subcores
    except Exception:
        return 2, 16


@functools.cache
def _build_gather(B, V, D):
    NC, NS = _sc_geometry()
    NW = NC * NS
    assert B % (8 * NW) == 0
    b_per_w = B // NW
    mesh = plsc.VectorSubcoreMesh(core_axis_name="c", subcore_axis_name="s")

    @functools.partial(
        pl.kernel,
        mesh=mesh,
        out_type=jax.ShapeDtypeStruct((B, D), jnp.float32),
        scratch_types=[
            pltpu.VMEM((b_per_w,), jnp.int32),
            pltpu.VMEM((b_per_w, D), jnp.float32),
            pltpu.SemaphoreType.DMA,
        ],
        compiler_params=pltpu.CompilerParams(use_tc_tiling_on_sc=False),
    )
    def gather_kernel(table_hbm, idx_hbm, out_hbm, idx_v, rows_v, sem):
        wid = lax.axis_index("s") * NC + lax.axis_index("c")
        base = wid * b_per_w
        pltpu.sync_copy(idx_hbm.at[pl.ds(base, b_per_w)], idx_v)
        pltpu.async_copy(table_hbm.at[idx_v], rows_v, sem).wait()
        pltpu.sync_copy(rows_v, out_hbm.at[pl.ds(base, b_per_w)])

    return gather_kernel


def kernel(g, h, r, norm, W):
    idx = h.reshape(-1)
    B = idx.shape[0]
    V, D = W.shape
    return _build_gather(B, V, D)(W, idx)


# speedup vs baseline: 3.7945x; 3.7945x over previous
import functools

import jax
import jax.numpy as jnp
from jax import lax
from jax.experimental import pallas as pl
from jax.experimental.pallas import tpu as pltpu
from jax.experimental.pallas import tpu_sc as plsc

_G = 16


def _sc_geometry():
    try:
        info = plsc.get_sparse_core_info()
        return info.num_cores, info.num_subcores
    except Exception:
        return 2, 16


@functools.cache
def _build_gather(B, V, D):
    NC, NS = _sc_geometry()
    NW = NC * NS
    assert B % (NW * _G) == 0
    b_per_w = B // NW
    n_groups = b_per_w // _G
    mesh = plsc.VectorSubcoreMesh(core_axis_name="c", subcore_axis_name="s")

    @functools.partial(
        pl.kernel,
        mesh=mesh,
        out_type=jax.ShapeDtypeStruct((D, B), jnp.float32),
        scratch_types=[
            pltpu.VMEM((b_per_w,), jnp.int32),
            pltpu.VMEM((_G, D, 128), jnp.float32),
            pltpu.VMEM((D, 128), jnp.float32),
            pltpu.SemaphoreType.DMA,
        ],
        compiler_params=pltpu.CompilerParams(
            use_tc_tiling_on_sc=True, needs_layout_passes=False
        ),
    )
    def gather_kernel(table_hbm, idx_hbm, out_hbm, idx_v, bufs_v, stage_v, sem):
        wid = lax.axis_index("s") * NC + lax.axis_index("c")
        base = wid * b_per_w
        pltpu.sync_copy(idx_hbm.at[pl.ds(base, b_per_w)], idx_v)
        iota0 = lax.iota(jnp.int32, _G)

        def group(g, carry):
            for sg in range(128 // _G):
                vec = idx_v[pl.ds(g * 128 + sg * _G, _G)]
                tc16 = lax.shift_right_logical(vec, 7)
                l16 = lax.bitwise_and(vec, 127)
                copies = []
                for t in range(_G):
                    start = pl.multiple_of(tc16[t] * 128, 128)
                    copies.append(
                        pltpu.async_copy(
                            table_hbm.at[:, pl.ds(start, 128)], bufs_v.at[t], sem
                        )
                    )
                for c in copies:
                    c.wait()
                for d in range(D):
                    d16 = jnp.full((_G,), d, jnp.int32)
                    stage_v[d, pl.ds(sg * _G, _G)] = plsc.load_gather(
                        bufs_v, [iota0, d16, l16]
                    )
            pltpu.sync_copy(
                stage_v, out_hbm.at[:, pl.ds(base + g * 128, 128)]
            )
            return carry

        lax.fori_loop(0, b_per_w // 128, group, 0)

    return gather_kernel


def kernel(g, h, r, norm, W):
    idx = h.reshape(-1)
    B = idx.shape[0]
    V, D = W.shape
    return _build_gather(B, V, D)(W.T, idx).T


# speedup vs baseline: 3.8699x; 1.0199x over previous
import functools

import jax
import jax.numpy as jnp
from jax import lax
from jax.experimental import pallas as pl
from jax.experimental.pallas import tpu as pltpu
from jax.experimental.pallas import tpu_sc as plsc

_P = 16
_NB = 3


def _sc_geometry():
    try:
        info = plsc.get_sparse_core_info()
        return info.num_cores, info.num_subcores
    except Exception:
        return 2, 16


@functools.cache
def _build_gather(B, V, D):
    NC, NS = _sc_geometry()
    NW = NC * NS
    assert B % (NW * 128) == 0 and D == 32
    b_per_w = B // NW
    n_groups = b_per_w // 128
    HD = D // 2
    steps = (128 // _P) * 2
    mesh = plsc.VectorSubcoreMesh(core_axis_name="c", subcore_axis_name="s")

    @functools.partial(
        pl.kernel,
        mesh=mesh,
        out_type=jax.ShapeDtypeStruct((D, B), jnp.float32),
        scratch_types=[
            pltpu.VMEM((b_per_w,), jnp.int32),
            pltpu.VMEM((_NB, _P, HD, 128), jnp.float32),
            pltpu.VMEM((D, 128), jnp.float32),
            pltpu.SemaphoreType.DMA,
            pltpu.SemaphoreType.DMA,
            pltpu.SemaphoreType.DMA,
        ],
        compiler_params=pltpu.CompilerParams(
            use_tc_tiling_on_sc=True, needs_layout_passes=False
        ),
    )
    def gather_kernel(table_hbm, idx_hbm, out_hbm, idx_v, bufs_v, stage_v, *sems):
        wid = lax.axis_index("s") * NC + lax.axis_index("c")
        base = wid * b_per_w
        pltpu.sync_copy(idx_hbm.at[pl.ds(base, b_per_w)], idx_v)
        iota16 = lax.iota(jnp.int32, 16)

        def load_pair(off):
            vec = idx_v[pl.ds(off, _P)]
            return lax.shift_right_logical(vec, 7), lax.bitwise_and(vec, 127)

        def fire(s, goff):
            bank = s % _NB
            fh = s % 2
            tc16, _ = load_pair(goff + (s // 2) * _P)
            cps = []
            for t in range(_P):
                start = pl.multiple_of(tc16[t] * 128, 128)
                cps.append(
                    pltpu.async_copy(
                        table_hbm.at[pl.ds(fh * HD, HD), pl.ds(start, 128)],
                        bufs_v.at[bank].at[t],
                        sems[bank],
                    )
                )
            return cps

        def group(g, carry):
            goff = g * 128
            pend = {}
            pend[0] = fire(0, goff)
            pend[1] = fire(1, goff)
            for s in range(steps):
                if s + 2 < steps:
                    pend[(s + 2) % _NB] = fire(s + 2, goff)
                bank = s % _NB
                for c in pend[bank]:
                    c.wait()
                _, l16 = load_pair(goff + (s // 2) * _P)
                dbase = (s % 2) * HD
                col = (s // 2) * _P
                for dd in range(HD):
                    d16 = jnp.full((16,), dd, jnp.int32)
                    stage_v[dbase + dd, pl.ds(col, _P)] = plsc.load_gather(
                        bufs_v.at[bank], [iota16, d16, l16]
                    )
            pltpu.sync_copy(stage_v, out_hbm.at[:, pl.ds(base + goff, 128)])
            return carry

        lax.fori_loop(0, n_groups, group, 0)

    return gather_kernel


def kernel(g, h, r, norm, W):
    idx = h.reshape(-1)
    B = idx.shape[0]
    V, D = W.shape
    return _build_gather(B, V, D)(W.T, idx).T
